# speedup vs baseline: 51.5227x; 51.5227x over previous
import jax
import jax.numpy as jnp
from jax.experimental import pallas as pl
from jax.experimental.pallas import tpu as pltpu

B, T, K = 512, 512, 128
START, STOP = 126, 127
NEG = -10000.0

B_BLK = 256
NB = B // B_BLK
T_BLK = 64
NT = T // T_BLK


def _crf_body(featsT_ref, tags_ref, tr_ref, trT_ref, logz_ref, gold_ref,
              alpha_s, ohprev_s, emit_s, trans_s):
    it = pl.program_id(1)
    ksub = jax.lax.broadcasted_iota(jnp.int32, (K, B_BLK), 0)

    @pl.when(it == 0)
    def _init():
        alpha_s[...] = jnp.where(ksub == START, 0.0, NEG)
        ohprev_s[...] = jnp.where(ksub == START, 1.0, 0.0)
        emit_s[...] = jnp.zeros((K, B_BLK), jnp.float32)
        trans_s[...] = jnp.zeros((K, B_BLK), jnp.float32)

    tr = tr_ref[...]
    trT = trT_ref[...]
    c = jnp.max(tr, axis=1, keepdims=True)
    et = jnp.exp(tr - c)

    def step(i, carry):
        featT = featsT_ref[i]
        tagrow = tags_ref[i]
        ohT = jnp.where(ksub == tagrow, 1.0, 0.0)

        alpha = alpha_s[...]
        m = jnp.max(alpha, axis=0, keepdims=True)
        w = jnp.exp(alpha - m)
        s = jnp.dot(et, w, preferred_element_type=jnp.float32)
        alpha_s[...] = m + c + jnp.log(s) + featT

        rows = jnp.dot(trT, ohT, preferred_element_type=jnp.float32)
        trans_s[...] += rows * ohprev_s[...]
        emit_s[...] += featT * ohT
        ohprev_s[...] = ohT
        return carry

    jax.lax.fori_loop(0, T_BLK, step, 0)

    @pl.when(it == NT - 1)
    def _fin():
        alpha = alpha_s[...]
        stop_row = tr_ref[STOP:STOP + 1, :]
        c2 = jnp.max(stop_row, axis=1, keepdims=True)
        estop = jnp.exp(stop_row - c2)
        m2 = jnp.max(alpha, axis=0, keepdims=True)
        w2 = jnp.exp(alpha - m2)
        z = jnp.dot(estop, w2, preferred_element_type=jnp.float32)
        logz_ref[...] = (m2 + c2 + jnp.log(z)).reshape(1, 1, B_BLK)

        stopv = jnp.dot(stop_row, ohprev_s[...],
                        preferred_element_type=jnp.float32)
        gold = jnp.sum(emit_s[...] + trans_s[...], axis=0, keepdims=True) + stopv
        gold_ref[...] = gold.reshape(1, 1, B_BLK)


def kernel(feats, tags, lengths, transitions):
    del lengths
    featsT = jnp.transpose(feats, (1, 2, 0))
    tagsT = jnp.transpose(tags.astype(jnp.int32), (1, 0)).reshape(T, 1, B)
    tr = transitions.astype(jnp.float32)
    trT = tr.T

    grid = (NB, NT)
    logz, gold = pl.pallas_call(
        _crf_body,
        grid=grid,
        in_specs=[
            pl.BlockSpec((T_BLK, K, B_BLK), lambda ib, it: (it, 0, ib)),
            pl.BlockSpec((T_BLK, 1, B_BLK), lambda ib, it: (it, 0, ib)),
            pl.BlockSpec((K, K), lambda ib, it: (0, 0)),
            pl.BlockSpec((K, K), lambda ib, it: (0, 0)),
        ],
        out_specs=[
            pl.BlockSpec((1, 1, B_BLK), lambda ib, it: (ib, 0, 0)),
            pl.BlockSpec((1, 1, B_BLK), lambda ib, it: (ib, 0, 0)),
        ],
        out_shape=[
            jax.ShapeDtypeStruct((NB, 1, B_BLK), jnp.float32),
            jax.ShapeDtypeStruct((NB, 1, B_BLK), jnp.float32),
        ],
        scratch_shapes=[pltpu.VMEM((K, B_BLK), jnp.float32)] * 4,
        compiler_params=pltpu.CompilerParams(
            dimension_semantics=("parallel", "arbitrary"),
        ),
    )(featsT, tagsT, tr, trT)
    return jnp.sum(logz) - jnp.sum(gold)
